# baseline (speedup 1.0000x reference)
import functools

import jax
import jax.numpy as jnp
from jax import lax
from jax.experimental import pallas as pl
from jax.experimental.pallas import tpu as pltpu
from jax.experimental.pallas import tpu_sc as plsc

_L = 16
_IDX_CHUNK = 128


@functools.cache
def _make_gather(B, V, D):
    info = plsc.get_sparse_core_info()
    NC, NS = info.num_cores, info.num_subcores
    NW = NC * NS
    b_per_w = B // NW
    n_groups = b_per_w // _L
    n_q = b_per_w // _IDX_CHUNK
    W = 2 * D
    mesh = plsc.VectorSubcoreMesh(core_axis_name="c", subcore_axis_name="s")

    @functools.partial(
        pl.kernel,
        mesh=mesh,
        out_type=jax.ShapeDtypeStruct((B, D), jnp.float32),
        scratch_types=[
            pltpu.VMEM((b_per_w,), jnp.int32),
            pltpu.VMEM((b_per_w,), jnp.int32),
            pltpu.VMEM((2, _IDX_CHUNK, W), jnp.float32),
            pltpu.VMEM((b_per_w, D), jnp.float32),
            pltpu.SemaphoreType.DMA,
            pltpu.SemaphoreType.DMA,
        ],
        compiler_params=pltpu.CompilerParams(needs_layout_passes=False),
    )
    def gather_kernel(
        table_hbm, idx_hbm, out_hbm, idx_v, pair_v, stage_v, rows_v, s0, s1
    ):
        wid = lax.axis_index("s") * NC + lax.axis_index("c")
        base = wid * b_per_w
        pltpu.sync_copy(idx_hbm.at[pl.ds(base, b_per_w)], idx_v)
        lane = lax.broadcasted_iota(jnp.int32, (_L,), 0)

        def make_pairs(g, carry):
            pair_v[pl.ds(g * _L, _L)] = idx_v[pl.ds(g * _L, _L)] >> 1
            return carry

        lax.fori_loop(0, n_groups, make_pairs, 0, unroll=4)

        sems = (s0, s1)

        def fire(q):
            return pltpu.async_copy(
                table_hbm.at[pair_v.at[pl.ds(q * _IDX_CHUNK, _IDX_CHUNK)]],
                stage_v.at[q % 2],
                sems[q % 2],
            )

        def select(q):
            def body(g, carry):
                vec = idx_v[pl.ds(q * _IDX_CHUNK + g * _L, _L)]
                offs = (vec & 1) * D
                for j in range(_L):
                    off = jnp.sum(jnp.where(lane == j, offs, 0))
                    i = g * _L + j
                    for k in range(D // _L):
                        rows_v[
                            q * _IDX_CHUNK + i, pl.ds(k * _L, _L)
                        ] = stage_v[q % 2, i, pl.ds(off + k * _L, _L)]
                return carry

            lax.fori_loop(0, _IDX_CHUNK // _L, body, 0, unroll=2)

        copies = [fire(0), fire(1)]
        for q in range(n_q):
            copies[q].wait()
            select(q)
            if q + 2 < n_q:
                copies.append(fire(q + 2))

        pltpu.sync_copy(rows_v, out_hbm.at[pl.ds(base, b_per_w)])

    return gather_kernel


def kernel(batch, table):
    B = batch.shape[0]
    V, D = table.shape
    table2 = table.reshape(V // 2, 2 * D)
    out = _make_gather(B, V, D)(table2, batch)
    return out[:, None, :]


# speedup vs baseline: 1.5494x; 1.5494x over previous
import functools

import jax
import jax.numpy as jnp
from jax import lax
from jax.experimental import pallas as pl
from jax.experimental.pallas import tpu as pltpu
from jax.experimental.pallas import tpu_sc as plsc

_L = 16
_DRAIN_EVERY = 4


@functools.cache
def _make_gather(B, V, D):
    info = plsc.get_sparse_core_info()
    NC, NS = info.num_cores, info.num_subcores
    NW = NC * NS
    b_per_w = B // NW
    n_groups = b_per_w // _L
    mesh = plsc.VectorSubcoreMesh(core_axis_name="c", subcore_axis_name="s")

    @functools.partial(
        pl.kernel,
        mesh=mesh,
        out_type=jax.ShapeDtypeStruct((B, D), jnp.float32),
        scratch_types=[
            pltpu.VMEM((b_per_w,), jnp.int32),
            pltpu.VMEM((b_per_w, D), jnp.float32),
            pltpu.SemaphoreType.DMA,
        ],
        compiler_params=pltpu.CompilerParams(needs_layout_passes=False),
    )
    def gather_kernel(table_hbm, idx_hbm, out_hbm, idx_v, rows_v, sem):
        wid = lax.axis_index("s") * NC + lax.axis_index("c")
        base = wid * b_per_w
        pltpu.sync_copy(idx_hbm.at[pl.ds(base, b_per_w)], idx_v)
        lane = lax.broadcasted_iota(jnp.int32, (_L,), 0)

        def body(g, carry):
            vec = idx_v[pl.ds(g * _L, _L)]
            for j in range(_L):
                row = jnp.sum(jnp.where(lane == j, vec, 0))
                pltpu.async_copy(
                    table_hbm.at[pl.ds(row, 1)],
                    rows_v.at[pl.ds(g * _L + j, 1)],
                    sem,
                )
            @pl.when(jnp.logical_and(g >= 2 * _DRAIN_EVERY - 1,
                                     g % _DRAIN_EVERY == _DRAIN_EVERY - 1))
            def _():
                pltpu.make_async_copy(
                    table_hbm.at[pl.ds(0, _DRAIN_EVERY * _L)],
                    rows_v.at[pl.ds(0, _DRAIN_EVERY * _L)],
                    sem,
                ).wait()

            return carry

        lax.fori_loop(0, n_groups, body, 0, unroll=2)
        pltpu.make_async_copy(
            table_hbm.at[pl.ds(0, _DRAIN_EVERY * _L)],
            rows_v.at[pl.ds(0, _DRAIN_EVERY * _L)],
            sem,
        ).wait()
        pltpu.sync_copy(rows_v, out_hbm.at[pl.ds(base, b_per_w)])

    return gather_kernel


def kernel(batch, table):
    B = batch.shape[0]
    V, D = table.shape
    out = _make_gather(B, V, D)(table, batch)
    return out[:, None, :]
